# baseline (speedup 1.0000x reference)
import jax
import jax.numpy as jnp
from jax.experimental import pallas as pl
from jax.experimental.pallas import tpu as pltpu

B, N, C_IN, C_OUT, EMB, RANK = 16, 512, 128, 128, 64, 16
BB = 16
_EPS = 1e-8
_LOG2 = 0.6931471805599453


def _gcn_body(x_ref, mem_ref, phy_ref, wg_ref, bg_ref, zu_ref, zv_ref,
              wp_ref, bp_ref, wm_ref, bm_ref,
              out_ref, elbo_ref, adj_s, pn_s):
    i = pl.program_id(0)
    _dot = lambda a, b: jnp.dot(a, b, preferred_element_type=jnp.float32)

    def _batches(adj, pn):
        for b in range(BB):
            xb = x_ref[b]
            xg = _dot(xb, wg_ref[...])
            xp = _dot(xb, wp_ref[...])
            att = _dot(adj, xg) + bg_ref[...]
            agg = _dot(pn, xp) + bp_ref[...]
            gate = jax.nn.sigmoid(_dot(mem_ref[b], wm_ref[...]) + bm_ref[...])
            out_ref[b] = att + gate * agg

    @pl.when(i == 0)
    def _init():
        phy = phy_ref[...]
        row = jax.lax.broadcasted_iota(jnp.int32, (N, N), 0)
        col = jax.lax.broadcasted_iota(jnp.int32, (N, N), 1)
        a_hat = phy + (row == col).astype(jnp.float32)
        deg_r = jnp.sum(a_hat, axis=1, keepdims=True)
        deg_c = jnp.sum(a_hat, axis=0, keepdims=True)
        sr = jax.lax.rsqrt(jnp.maximum(deg_r, 1.0))
        sc = jax.lax.rsqrt(jnp.maximum(deg_c, 1.0))
        adj = sr * a_hat * sc
        adj_s[...] = adj

        logits = jax.lax.dot_general(
            zu_ref[...], zv_ref[...], (((1,), (1,)), ((), ())),
            preferred_element_type=jnp.float32)
        p = jax.nn.sigmoid(logits)
        pn = p / (jnp.sum(p, axis=1, keepdims=True) + _EPS)
        pn_s[...] = pn
        elbo_ref[...] = (jnp.mean((phy - p) * logits) - _LOG2)[None, None]
        _batches(adj, pn)

    @pl.when(i != 0)
    def _steady():
        _batches(adj_s[...], pn_s[...])


def kernel(x, memory, phy_graph, W_gcn, b_gcn, Z_u, Z_v, W_pg, b_pg, W_mem, b_mem):
    bg = b_gcn.reshape(1, C_OUT)
    bp = b_pg.reshape(1, C_OUT)
    bm = b_mem.reshape(1, C_OUT)
    wg, wp, wm = W_gcn, W_pg, W_mem

    const = lambda shape: pl.BlockSpec(shape, lambda i: (0,) * len(shape))
    out, elbo = pl.pallas_call(
        _gcn_body,
        grid=(B // BB,),
        in_specs=[
            pl.BlockSpec((BB, N, C_IN), lambda i: (i, 0, 0)),
            pl.BlockSpec((BB, N, EMB), lambda i: (i, 0, 0)),
            const((N, N)),
            const((C_IN, C_OUT)),
            const((1, C_OUT)),
            const((N, RANK)),
            const((N, RANK)),
            const((C_IN, C_OUT)),
            const((1, C_OUT)),
            const((EMB, C_OUT)),
            const((1, C_OUT)),
        ],
        out_specs=[
            pl.BlockSpec((BB, N, C_OUT), lambda i: (i, 0, 0)),
            pl.BlockSpec((1, 1), lambda i: (0, 0)),
        ],
        out_shape=[
            jax.ShapeDtypeStruct((B, N, C_OUT), jnp.float32),
            jax.ShapeDtypeStruct((1, 1), jnp.float32),
        ],
        scratch_shapes=[
            pltpu.VMEM((N, N), jnp.float32),
            pltpu.VMEM((N, N), jnp.float32),
        ],
        compiler_params=pltpu.CompilerParams(
            dimension_semantics=("arbitrary",)),
    )(x, memory, phy_graph, wg, bg, Z_u, Z_v, wp, bp, wm, bm)
    return out, elbo[0, 0]


# speedup vs baseline: 1.0585x; 1.0585x over previous
import jax
import jax.numpy as jnp
from jax.experimental import pallas as pl
from jax.experimental.pallas import tpu as pltpu

B, N, C_IN, C_OUT, EMB, RANK = 16, 512, 128, 128, 64, 16
BB = 8
_EPS = 1e-8
_LOG2 = 0.6931471805599453


def _gcn_body(x_ref, mem_ref, phy_ref, wg_ref, bg_ref, zu_ref, zv_ref,
              wp_ref, bp_ref, wm_ref, bm_ref,
              out_ref, elbo_ref, adj_s, pn_s):
    i = pl.program_id(0)
    _dot = lambda a, b: jnp.dot(a, b, preferred_element_type=jnp.float32)

    def _batches(adj, pn):
        for b in range(BB):
            xb = x_ref[b]
            xg = _dot(xb, wg_ref[...])
            xp = _dot(xb, wp_ref[...])
            att = _dot(adj, xg) + bg_ref[...]
            agg = _dot(pn, xp) + bp_ref[...]
            gate = jax.nn.sigmoid(_dot(mem_ref[b], wm_ref[...]) + bm_ref[...])
            out_ref[b] = att + gate * agg

    @pl.when(i == 0)
    def _init():
        phy = phy_ref[...]
        row = jax.lax.broadcasted_iota(jnp.int32, (N, N), 0)
        col = jax.lax.broadcasted_iota(jnp.int32, (N, N), 1)
        a_hat = phy + (row == col).astype(jnp.float32)
        deg_r = jnp.sum(a_hat, axis=1, keepdims=True)
        deg_c = jnp.sum(a_hat, axis=0, keepdims=True)
        sr = jax.lax.rsqrt(jnp.maximum(deg_r, 1.0))
        sc = jax.lax.rsqrt(jnp.maximum(deg_c, 1.0))
        adj = sr * a_hat * sc
        adj_s[...] = adj

        logits = jax.lax.dot_general(
            zu_ref[...], zv_ref[...], (((1,), (1,)), ((), ())),
            preferred_element_type=jnp.float32)
        p = jax.nn.sigmoid(logits)
        pn = p / (jnp.sum(p, axis=1, keepdims=True) + _EPS)
        pn_s[...] = pn
        elbo_ref[...] = (jnp.mean((phy - p) * logits) - _LOG2)[None, None]
        _batches(adj, pn)

    @pl.when(i != 0)
    def _steady():
        _batches(adj_s[...], pn_s[...])


def kernel(x, memory, phy_graph, W_gcn, b_gcn, Z_u, Z_v, W_pg, b_pg, W_mem, b_mem):
    bg = b_gcn.reshape(1, C_OUT)
    bp = b_pg.reshape(1, C_OUT)
    bm = b_mem.reshape(1, C_OUT)
    wg, wp, wm = W_gcn, W_pg, W_mem

    const = lambda shape: pl.BlockSpec(shape, lambda i: (0,) * len(shape))
    out, elbo = pl.pallas_call(
        _gcn_body,
        grid=(B // BB,),
        in_specs=[
            pl.BlockSpec((BB, N, C_IN), lambda i: (i, 0, 0)),
            pl.BlockSpec((BB, N, EMB), lambda i: (i, 0, 0)),
            const((N, N)),
            const((C_IN, C_OUT)),
            const((1, C_OUT)),
            const((N, RANK)),
            const((N, RANK)),
            const((C_IN, C_OUT)),
            const((1, C_OUT)),
            const((EMB, C_OUT)),
            const((1, C_OUT)),
        ],
        out_specs=[
            pl.BlockSpec((BB, N, C_OUT), lambda i: (i, 0, 0)),
            pl.BlockSpec((1, 1), lambda i: (0, 0)),
        ],
        out_shape=[
            jax.ShapeDtypeStruct((B, N, C_OUT), jnp.float32),
            jax.ShapeDtypeStruct((1, 1), jnp.float32),
        ],
        scratch_shapes=[
            pltpu.VMEM((N, N), jnp.float32),
            pltpu.VMEM((N, N), jnp.float32),
        ],
        compiler_params=pltpu.CompilerParams(
            dimension_semantics=("arbitrary",)),
    )(x, memory, phy_graph, wg, bg, Z_u, Z_v, wp, bp, wm, bm)
    return out, elbo[0, 0]
